# baseline (speedup 1.0000x reference)
import functools

import jax
import jax.numpy as jnp
from jax import lax
from jax.experimental import pallas as pl
from jax.experimental.pallas import tpu as pltpu
from jax.experimental.pallas import tpu_sc as plsc

_N_EXP = 64
_D = 2048
_N_TOK = 8192
_NW = 32
_TPW = _N_TOK // _NW
_L = 16


_BT = 2048
_WPB = _BT // _TPW


def _gemm_body(x_ref, w_ref, out_ref):
    for j in range(_WPB):
        out_ref[j] = lax.dot_general(
            w_ref[...], x_ref[pl.ds(j * _TPW, _TPW), :],
            (((1,), (1,)), ((), ())),
            preferred_element_type=jnp.float32,
        )


def _gemm(x, w_gate):
    return pl.pallas_call(
        _gemm_body,
        grid=(_N_TOK // _BT,),
        in_specs=[
            pl.BlockSpec((_BT, _D), lambda i: (i, 0)),
            pl.BlockSpec((_N_EXP, _D), lambda i: (0, 0)),
        ],
        out_specs=pl.BlockSpec((_WPB, _N_EXP, _TPW), lambda i: (i, 0, 0)),
        out_shape=jax.ShapeDtypeStruct((_NW, _N_EXP, _TPW), jnp.float32),
    )(x, w_gate)


@functools.cache
def _make_router():
    mesh = plsc.VectorSubcoreMesh(core_axis_name="c", subcore_axis_name="s")
    return functools.partial(
        pl.kernel,
        mesh=mesh,
        out_type=[
            jax.ShapeDtypeStruct((_N_TOK,), jnp.int32),
            jax.ShapeDtypeStruct((_N_TOK,), jnp.int32),
            jax.ShapeDtypeStruct((_N_TOK,), jnp.float32),
            jax.ShapeDtypeStruct((_N_TOK,), jnp.float32),
        ],
        scratch_types=[
            pltpu.VMEM((_N_EXP, _TPW), jnp.float32),
            pltpu.VMEM((_TPW,), jnp.int32),
            pltpu.VMEM((_TPW,), jnp.int32),
            pltpu.VMEM((_TPW,), jnp.float32),
            pltpu.VMEM((_TPW,), jnp.float32),
        ],
    )(_router_body)


def _router_body(logits_hbm, i1_hbm, i2_hbm, g1_hbm, g2_hbm,
                 lg_v, i1_v, i2_v, g1_v, g2_v):
    wid = lax.axis_index("s") * 2 + lax.axis_index("c")
    base = wid * _TPW
    pltpu.sync_copy(logits_hbm.at[wid], lg_v)

    def group(g, carry):
        off = g * _L
        neg_inf = jnp.full((_L,), -jnp.inf, jnp.float32)
        m1 = neg_inf
        m2 = neg_inf
        zero_i = jnp.zeros((_L,), jnp.int32)
        i1 = zero_i
        i2 = zero_i
        for e in range(_N_EXP):
            v = lg_v[e, pl.ds(off, _L)]
            is1 = v > m1
            is2 = v > m2
            e_vec = jnp.full((_L,), e, jnp.int32)
            t_i2 = jnp.where(is2, e_vec, i2)
            t_m2 = jnp.where(is2, v, m2)
            i2 = jnp.where(is1, i1, t_i2)
            m2 = jnp.where(is1, m1, t_m2)
            i1 = jnp.where(is1, e_vec, i1)
            m1 = jnp.where(is1, v, m1)
        s = jnp.zeros((_L,), jnp.float32)
        for e in range(_N_EXP):
            v = lg_v[e, pl.ds(off, _L)]
            s = s + jnp.exp(v - m1)
        inv_s = jnp.float32(1.0) / s
        i1_v[pl.ds(off, _L)] = i1
        i2_v[pl.ds(off, _L)] = i2
        g1_v[pl.ds(off, _L)] = inv_s
        g2_v[pl.ds(off, _L)] = jnp.exp(m2 - m1) * inv_s
        return carry

    lax.fori_loop(0, _TPW // _L, group, 0)

    pltpu.sync_copy(i1_v, i1_hbm.at[pl.ds(base, _TPW)])
    pltpu.sync_copy(i2_v, i2_hbm.at[pl.ds(base, _TPW)])
    pltpu.sync_copy(g1_v, g1_hbm.at[pl.ds(base, _TPW)])
    pltpu.sync_copy(g2_v, g2_hbm.at[pl.ds(base, _TPW)])


def kernel(x, w_gate):
    logits = _gemm(x, w_gate)
    i1, i2, g1, g2 = _make_router()(logits)
    top_k_indices = jnp.stack((i1, i2), axis=1)
    top_k_gates = jnp.stack((g1, g2), axis=1)
    return (top_k_indices, top_k_gates)


# speedup vs baseline: 1.0367x; 1.0367x over previous
import functools

import jax
import jax.numpy as jnp
from jax import lax
from jax.experimental import pallas as pl
from jax.experimental.pallas import tpu as pltpu
from jax.experimental.pallas import tpu_sc as plsc

_N_EXP = 64
_D = 2048
_N_TOK = 8192
_NW = 32
_TPW = _N_TOK // _NW
_L = 16


_BT = 1024
_WPB = _BT // _TPW


def _gemm_body(x_ref, w_ref, out_ref):
    for j in range(_WPB):
        out_ref[j] = lax.dot_general(
            w_ref[...], x_ref[pl.ds(j * _TPW, _TPW), :],
            (((1,), (1,)), ((), ())),
            preferred_element_type=jnp.float32,
        )


def _gemm(x, w_gate):
    return pl.pallas_call(
        _gemm_body,
        grid=(_N_TOK // _BT,),
        in_specs=[
            pl.BlockSpec((_BT, _D), lambda i: (i, 0)),
            pl.BlockSpec((_N_EXP, _D), lambda i: (0, 0)),
        ],
        out_specs=pl.BlockSpec((_WPB, _N_EXP, _TPW), lambda i: (i, 0, 0)),
        out_shape=jax.ShapeDtypeStruct((_NW, _N_EXP, _TPW), jnp.float32),
    )(x, w_gate)


@functools.cache
def _make_router():
    mesh = plsc.VectorSubcoreMesh(core_axis_name="c", subcore_axis_name="s")
    return functools.partial(
        pl.kernel,
        mesh=mesh,
        out_type=[
            jax.ShapeDtypeStruct((_N_TOK,), jnp.int32),
            jax.ShapeDtypeStruct((_N_TOK,), jnp.int32),
            jax.ShapeDtypeStruct((_N_TOK,), jnp.float32),
            jax.ShapeDtypeStruct((_N_TOK,), jnp.float32),
        ],
        scratch_types=[
            pltpu.VMEM((_N_EXP, _TPW), jnp.float32),
            pltpu.VMEM((_TPW,), jnp.int32),
            pltpu.VMEM((_TPW,), jnp.int32),
            pltpu.VMEM((_TPW,), jnp.float32),
            pltpu.VMEM((_TPW,), jnp.float32),
        ],
    )(_router_body)


def _router_body(logits_hbm, i1_hbm, i2_hbm, g1_hbm, g2_hbm,
                 lg_v, i1_v, i2_v, g1_v, g2_v):
    wid = lax.axis_index("s") * 2 + lax.axis_index("c")
    base = wid * _TPW
    pltpu.sync_copy(logits_hbm.at[wid], lg_v)

    def group(g, carry):
        off = g * _L
        neg_inf = jnp.full((_L,), -jnp.inf, jnp.float32)
        m1 = neg_inf
        m2 = neg_inf
        zero_i = jnp.zeros((_L,), jnp.int32)
        i1 = zero_i
        i2 = zero_i
        for e in range(_N_EXP):
            v = lg_v[e, pl.ds(off, _L)]
            is1 = v > m1
            is2 = v > m2
            e_vec = jnp.full((_L,), e, jnp.int32)
            t_i2 = jnp.where(is2, e_vec, i2)
            t_m2 = jnp.where(is2, v, m2)
            i2 = jnp.where(is1, i1, t_i2)
            m2 = jnp.where(is1, m1, t_m2)
            i1 = jnp.where(is1, e_vec, i1)
            m1 = jnp.where(is1, v, m1)
        s = jnp.zeros((_L,), jnp.float32)
        for e in range(_N_EXP):
            v = lg_v[e, pl.ds(off, _L)]
            s = s + jnp.exp(v - m1)
        inv_s = jnp.float32(1.0) / s
        i1_v[pl.ds(off, _L)] = i1
        i2_v[pl.ds(off, _L)] = i2
        g1_v[pl.ds(off, _L)] = inv_s
        g2_v[pl.ds(off, _L)] = jnp.exp(m2 - m1) * inv_s
        return carry

    lax.fori_loop(0, _TPW // _L, group, 0)

    pltpu.sync_copy(i1_v, i1_hbm.at[pl.ds(base, _TPW)])
    pltpu.sync_copy(i2_v, i2_hbm.at[pl.ds(base, _TPW)])
    pltpu.sync_copy(g1_v, g1_hbm.at[pl.ds(base, _TPW)])
    pltpu.sync_copy(g2_v, g2_hbm.at[pl.ds(base, _TPW)])


def kernel(x, w_gate):
    logits = _gemm(x, w_gate)
    i1, i2, g1, g2 = _make_router()(logits)
    top_k_indices = jnp.stack((i1, i2), axis=1)
    top_k_gates = jnp.stack((g1, g2), axis=1)
    return (top_k_indices, top_k_gates)
